# baseline (speedup 1.0000x reference)
import functools

import jax
import jax.numpy as jnp
from jax import lax
from jax.experimental import pallas as pl
from jax.experimental.pallas import tpu as pltpu
from jax.experimental.pallas import tpu_sc as plsc

NC = 2
NS = 16
NW = NC * NS
CHUNK = 128
NBUF = 4


def _sc_degree(dst3, ones_d, zeros_d, np_):
    kch = dst3.shape[1]
    d = ones_d.shape[1]
    stripe = np_ // NS
    mesh = plsc.VectorSubcoreMesh(core_axis_name="c", subcore_axis_name="s")

    @functools.partial(
        pl.kernel,
        out_type=jax.ShapeDtypeStruct((NC, np_, d), jnp.float32),
        mesh=mesh,
        scratch_types=[
            pltpu.VMEM((kch, CHUNK), jnp.int32),
            pltpu.VMEM((CHUNK, d), jnp.float32),
            pltpu.VMEM_SHARED((np_, d), jnp.float32),
        ],
    )
    def deg_kernel(dst_hbm, ones_hbm, zeros_hbm, out_hbm, dst_v, ones_v, acc):
        c = lax.axis_index("c")
        s = lax.axis_index("s")
        w = c * NS + s
        pltpu.sync_copy(zeros_hbm.at[pl.ds(s * stripe, stripe)],
                        acc.at[pl.ds(s * stripe, stripe)])
        pltpu.sync_copy(dst_hbm.at[w], dst_v)
        pltpu.sync_copy(ones_hbm, ones_v)
        plsc.subcore_barrier()

        def body(k, carry):
            pltpu.sync_copy(ones_v, acc.at[dst_v.at[k]], add=True)
            return carry

        lax.fori_loop(0, kch, body, 0)
        plsc.subcore_barrier()
        pltpu.sync_copy(acc.at[pl.ds(s * stripe, stripe)],
                        out_hbm.at[c, pl.ds(s * stripe, stripe)])

    return deg_kernel(dst3, ones_d, zeros_d)


GI = 16


def _sc_aggregate(src3, dst3, table, zeros_d, np_):
    kch = src3.shape[1]
    d = table.shape[1]
    stripe = np_ // NS
    ng = kch // GI
    mesh = plsc.VectorSubcoreMesh(core_axis_name="c", subcore_axis_name="s")

    @functools.partial(
        pl.kernel,
        out_type=jax.ShapeDtypeStruct((NC, np_, d), jnp.float32),
        mesh=mesh,
        scratch_types=[
            pltpu.VMEM((2, GI, CHUNK), jnp.int32),
            pltpu.VMEM((2, GI, CHUNK), jnp.int32),
            pltpu.VMEM((2, CHUNK, d), jnp.float32),
            pltpu.VMEM_SHARED((np_, d), jnp.float32),
            pltpu.SemaphoreType.DMA((2,)),
            pltpu.SemaphoreType.DMA,
            pltpu.SemaphoreType.DMA,
        ],
    )
    def agg_kernel(src_hbm, dst_hbm, tab_hbm, zeros_hbm, out_hbm,
                   src_v, dst_v, rows_v, acc, gsem, issem, idsem):
        c = lax.axis_index("c")
        s = lax.axis_index("s")
        w = c * NS + s
        pltpu.sync_copy(zeros_hbm.at[pl.ds(s * stripe, stripe)],
                        acc.at[pl.ds(s * stripe, stripe)])
        pltpu.sync_copy(src_hbm.at[w, pl.ds(0, GI)], src_v.at[0])
        pltpu.sync_copy(dst_hbm.at[w, pl.ds(0, GI)], dst_v.at[0])
        plsc.subcore_barrier()

        def group(g, carry):
            gs = lax.rem(g, 2)

            @pl.when(g > 0)
            def _():
                pltpu.make_async_copy(src_hbm.at[w, pl.ds(g * GI, GI)],
                                      src_v.at[gs], issem).wait()
                pltpu.make_async_copy(dst_hbm.at[w, pl.ds(g * GI, GI)],
                                      dst_v.at[gs], idsem).wait()

            @pl.when(g + 1 < ng)
            def _():
                gn = lax.rem(g + 1, 2)
                pltpu.async_copy(src_hbm.at[w, pl.ds((g + 1) * GI, GI)],
                                 src_v.at[gn], issem)
                pltpu.async_copy(dst_hbm.at[w, pl.ds((g + 1) * GI, GI)],
                                 dst_v.at[gn], idsem)

            for b in range(2):
                pltpu.async_copy(tab_hbm.at[src_v.at[gs, b]], rows_v.at[b],
                                 gsem.at[b])

            def pair(j2, carry):
                for b in range(2):
                    j = j2 * 2 + b
                    pltpu.make_async_copy(tab_hbm.at[src_v.at[gs, j]],
                                          rows_v.at[b], gsem.at[b]).wait()
                    pltpu.sync_copy(rows_v.at[b], acc.at[dst_v.at[gs, j]],
                                    add=True)

                    @pl.when(j2 < GI // 2 - 1)
                    def _():
                        pltpu.async_copy(tab_hbm.at[src_v.at[gs, j + 2]],
                                         rows_v.at[b], gsem.at[b])
                return carry

            lax.fori_loop(0, GI // 2, pair, 0)
            return carry

        lax.fori_loop(0, ng, group, 0)
        plsc.subcore_barrier()
        pltpu.sync_copy(acc.at[pl.ds(s * stripe, stripe)],
                        out_hbm.at[c, pl.ds(s * stripe, stripe)])

    return agg_kernel(src3, dst3, table, zeros_d)


def _dis(deg_ref):
    d = deg_ref[0, :, 0:1] + deg_ref[1, :, 0:1] + 1.0
    return lax.rsqrt(d)


def _scale_body(deg_ref, x_ref, u_ref):
    u_ref[...] = x_ref[...] * _dis(deg_ref)


def _mid_body(deg_ref, agg_ref, u_ref, w1_ref, b1_ref, w2_ref, v_ref):
    r = _dis(deg_ref)
    a = (agg_ref[0] + agg_ref[1] + u_ref[...]) * r
    z = lax.dot_general(a, w1_ref[...], (((1,), (0,)), ((), ())),
                        precision=lax.Precision.HIGHEST,
                        preferred_element_type=jnp.float32) + b1_ref[...]
    z = jnp.maximum(z, 0.0)
    h2 = lax.dot_general(z, w2_ref[...], (((1,), (0,)), ((), ())),
                         precision=lax.Precision.HIGHEST,
                         preferred_element_type=jnp.float32)
    v_ref[...] = h2 * r


def _out_body(deg_ref, agg_ref, v_ref, b2_ref, o_ref):
    y = (agg_ref[0] + agg_ref[1] + v_ref[...]) * _dis(deg_ref) + b2_ref[...]
    m = jnp.max(y, axis=1, keepdims=True)
    e = jnp.exp(y - m)
    o_ref[...] = y - m - jnp.log(jnp.sum(e, axis=1, keepdims=True))


def _tc_scale(deg, x, bn):
    n, d = x.shape
    return pl.pallas_call(
        _scale_body,
        grid=(n // bn,),
        in_specs=[pl.BlockSpec((NC, bn, 128), lambda i: (0, i, 0)),
                  pl.BlockSpec((bn, d), lambda i: (i, 0))],
        out_specs=pl.BlockSpec((bn, d), lambda i: (i, 0)),
        out_shape=jax.ShapeDtypeStruct((n, d), jnp.float32),
    )(deg, x)


def _tc_mid(deg, agg, u, w1, b1, w2, bn):
    n, d_in = u.shape
    d_hid, d_out2 = w2.shape
    return pl.pallas_call(
        _mid_body,
        grid=(n // bn,),
        in_specs=[pl.BlockSpec((NC, bn, 128), lambda i: (0, i, 0)),
                  pl.BlockSpec((NC, bn, d_in), lambda i: (0, i, 0)),
                  pl.BlockSpec((bn, d_in), lambda i: (i, 0)),
                  pl.BlockSpec(w1.shape, lambda i: (0, 0)),
                  pl.BlockSpec(b1.shape, lambda i: (0, 0)),
                  pl.BlockSpec(w2.shape, lambda i: (0, 0))],
        out_specs=pl.BlockSpec((bn, d_out2), lambda i: (i, 0)),
        out_shape=jax.ShapeDtypeStruct((n, d_out2), jnp.float32),
    )(deg, agg, u, w1, b1, w2)


def _tc_out(deg, agg, v, b2, bn):
    n, d = v.shape
    return pl.pallas_call(
        _out_body,
        grid=(n // bn,),
        in_specs=[pl.BlockSpec((NC, bn, 128), lambda i: (0, i, 0)),
                  pl.BlockSpec((NC, bn, d), lambda i: (0, i, 0)),
                  pl.BlockSpec((bn, d), lambda i: (i, 0)),
                  pl.BlockSpec(b2.shape, lambda i: (0, 0))],
        out_specs=pl.BlockSpec((bn, d), lambda i: (i, 0)),
        out_shape=jax.ShapeDtypeStruct((n, d), jnp.float32),
    )(deg, agg, v, b2)


def kernel(x, edge_index, W1, b1, W2, b2):
    n = x.shape[0]
    e = edge_index.shape[1]
    np_ = ((n + 1 + 8 * NS - 1) // (8 * NS)) * (8 * NS)
    grain = NW * CHUNK * GI
    ep = ((e + grain - 1) // grain) * grain
    kch = ep // (NW * CHUNK)

    pad = ep - e
    pad_dst = n + jnp.arange(pad, dtype=jnp.int32) % (np_ - n)
    src = jnp.concatenate(
        [edge_index[0], jnp.zeros((pad,), jnp.int32)]).reshape(NW, kch, CHUNK)
    dst = jnp.concatenate(
        [edge_index[1], pad_dst]).reshape(NW, kch, CHUNK)

    ones_d = jnp.ones((CHUNK, x.shape[1]), jnp.float32)
    zeros_d = jnp.zeros((np_, x.shape[1]), jnp.float32)

    bn = 1000 if n % 1000 == 0 else 8 * (n // 8)

    deg = _sc_degree(dst, ones_d, zeros_d, np_)
    u = _tc_scale(deg, x, bn)
    agg1 = _sc_aggregate(src, dst, u, zeros_d, np_)
    v = _tc_mid(deg, agg1, u, W1, b1.reshape(1, -1), W2, bn)
    agg2 = _sc_aggregate(src, dst, v, zeros_d, np_)
    return _tc_out(deg, agg2, v, b2.reshape(1, -1), bn)


# speedup vs baseline: 1.0185x; 1.0185x over previous
import functools

import jax
import jax.numpy as jnp
from jax import lax
from jax.experimental import pallas as pl
from jax.experimental.pallas import tpu as pltpu
from jax.experimental.pallas import tpu_sc as plsc

NC = 2
NS = 16
NW = NC * NS
CHUNK = 128
NBUF = 4


def _sc_degree(dst3, ones_d, zeros_d, np_):
    kch = dst3.shape[1]
    d = ones_d.shape[1]
    stripe = np_ // NS
    mesh = plsc.VectorSubcoreMesh(core_axis_name="c", subcore_axis_name="s")

    @functools.partial(
        pl.kernel,
        out_type=jax.ShapeDtypeStruct((NC, np_, d), jnp.float32),
        mesh=mesh,
        scratch_types=[
            pltpu.VMEM((kch, CHUNK), jnp.int32),
            pltpu.VMEM((CHUNK, d), jnp.float32),
            pltpu.VMEM_SHARED((np_, d), jnp.float32),
        ],
    )
    def deg_kernel(dst_hbm, ones_hbm, zeros_hbm, out_hbm, dst_v, ones_v, acc):
        c = lax.axis_index("c")
        s = lax.axis_index("s")
        w = c * NS + s
        pltpu.sync_copy(zeros_hbm.at[pl.ds(s * stripe, stripe)],
                        acc.at[pl.ds(s * stripe, stripe)])
        pltpu.sync_copy(dst_hbm.at[w], dst_v)
        pltpu.sync_copy(ones_hbm, ones_v)
        plsc.subcore_barrier()

        def body(k, carry):
            pltpu.sync_copy(ones_v, acc.at[dst_v.at[k]], add=True)
            return carry

        lax.fori_loop(0, kch, body, 0)
        plsc.subcore_barrier()
        pltpu.sync_copy(acc.at[pl.ds(s * stripe, stripe)],
                        out_hbm.at[c, pl.ds(s * stripe, stripe)])

    return deg_kernel(dst3, ones_d, zeros_d)


GI = 16


def _sc_aggregate(src3, dst3, table, zeros_d, np_):
    kch = src3.shape[1]
    d = table.shape[1]
    stripe = np_ // NS
    ng = kch // GI
    mesh = plsc.VectorSubcoreMesh(core_axis_name="c", subcore_axis_name="s")

    @functools.partial(
        pl.kernel,
        out_type=jax.ShapeDtypeStruct((NC, np_, d), jnp.float32),
        mesh=mesh,
        scratch_types=[
            pltpu.VMEM((2, GI, CHUNK), jnp.int32),
            pltpu.VMEM((2, GI, CHUNK), jnp.int32),
            pltpu.VMEM((2, CHUNK, d), jnp.float32),
            pltpu.VMEM_SHARED((np_, d), jnp.float32),
            pltpu.SemaphoreType.DMA((2,)),
            pltpu.SemaphoreType.DMA,
            pltpu.SemaphoreType.DMA,
        ],
    )
    def agg_kernel(src_hbm, dst_hbm, tab_hbm, zeros_hbm, out_hbm,
                   src_v, dst_v, rows_v, acc, gsem, issem, idsem):
        c = lax.axis_index("c")
        s = lax.axis_index("s")
        w = (1 - c) * NS + s
        pltpu.sync_copy(zeros_hbm.at[pl.ds(s * stripe, stripe)],
                        acc.at[pl.ds(s * stripe, stripe)])
        pltpu.sync_copy(src_hbm.at[w, pl.ds(0, GI)], src_v.at[0])
        pltpu.sync_copy(dst_hbm.at[w, pl.ds(0, GI)], dst_v.at[0])
        plsc.subcore_barrier()

        def group(g, carry):
            gs = lax.rem(g, 2)

            @pl.when(g > 0)
            def _():
                pltpu.make_async_copy(src_hbm.at[w, pl.ds(g * GI, GI)],
                                      src_v.at[gs], issem).wait()
                pltpu.make_async_copy(dst_hbm.at[w, pl.ds(g * GI, GI)],
                                      dst_v.at[gs], idsem).wait()

            @pl.when(g + 1 < ng)
            def _():
                gn = lax.rem(g + 1, 2)
                pltpu.async_copy(src_hbm.at[w, pl.ds((g + 1) * GI, GI)],
                                 src_v.at[gn], issem)
                pltpu.async_copy(dst_hbm.at[w, pl.ds((g + 1) * GI, GI)],
                                 dst_v.at[gn], idsem)

            for b in range(2):
                pltpu.async_copy(tab_hbm.at[src_v.at[gs, b]], rows_v.at[b],
                                 gsem.at[b])

            def pair(j2, carry):
                for b in range(2):
                    j = j2 * 2 + b
                    pltpu.make_async_copy(tab_hbm.at[src_v.at[gs, j]],
                                          rows_v.at[b], gsem.at[b]).wait()
                    pltpu.sync_copy(rows_v.at[b], acc.at[dst_v.at[gs, j]],
                                    add=True)

                    @pl.when(j2 < GI // 2 - 1)
                    def _():
                        pltpu.async_copy(tab_hbm.at[src_v.at[gs, j + 2]],
                                         rows_v.at[b], gsem.at[b])
                return carry

            lax.fori_loop(0, GI // 2, pair, 0)
            return carry

        lax.fori_loop(0, ng, group, 0)
        plsc.subcore_barrier()
        pltpu.sync_copy(acc.at[pl.ds(s * stripe, stripe)],
                        out_hbm.at[c, pl.ds(s * stripe, stripe)])

    return agg_kernel(src3, dst3, table, zeros_d)


def _dis(deg_ref):
    d = deg_ref[0, :, 0:1] + deg_ref[1, :, 0:1] + 1.0
    return lax.rsqrt(d)


def _scale_body(deg_ref, x_ref, u_ref):
    u_ref[...] = x_ref[...] * _dis(deg_ref)


def _mid_body(deg_ref, agg_ref, u_ref, w1_ref, b1_ref, w2_ref, v_ref):
    r = _dis(deg_ref)
    a = (agg_ref[0] + agg_ref[1] + u_ref[...]) * r
    z = lax.dot_general(a, w1_ref[...], (((1,), (0,)), ((), ())),
                        precision=lax.Precision.HIGHEST,
                        preferred_element_type=jnp.float32) + b1_ref[...]
    z = jnp.maximum(z, 0.0)
    h2 = lax.dot_general(z, w2_ref[...], (((1,), (0,)), ((), ())),
                         precision=lax.Precision.HIGHEST,
                         preferred_element_type=jnp.float32)
    v_ref[...] = h2 * r


def _out_body(deg_ref, agg_ref, v_ref, b2_ref, o_ref):
    y = (agg_ref[0] + agg_ref[1] + v_ref[...]) * _dis(deg_ref) + b2_ref[...]
    m = jnp.max(y, axis=1, keepdims=True)
    e = jnp.exp(y - m)
    o_ref[...] = y - m - jnp.log(jnp.sum(e, axis=1, keepdims=True))


def _tc_scale(deg, x, bn):
    n, d = x.shape
    return pl.pallas_call(
        _scale_body,
        grid=(n // bn,),
        in_specs=[pl.BlockSpec((NC, bn, 128), lambda i: (0, i, 0)),
                  pl.BlockSpec((bn, d), lambda i: (i, 0))],
        out_specs=pl.BlockSpec((bn, d), lambda i: (i, 0)),
        out_shape=jax.ShapeDtypeStruct((n, d), jnp.float32),
    )(deg, x)


def _tc_mid(deg, agg, u, w1, b1, w2, bn):
    n, d_in = u.shape
    d_hid, d_out2 = w2.shape
    return pl.pallas_call(
        _mid_body,
        grid=(n // bn,),
        in_specs=[pl.BlockSpec((NC, bn, 128), lambda i: (0, i, 0)),
                  pl.BlockSpec((NC, bn, d_in), lambda i: (0, i, 0)),
                  pl.BlockSpec((bn, d_in), lambda i: (i, 0)),
                  pl.BlockSpec(w1.shape, lambda i: (0, 0)),
                  pl.BlockSpec(b1.shape, lambda i: (0, 0)),
                  pl.BlockSpec(w2.shape, lambda i: (0, 0))],
        out_specs=pl.BlockSpec((bn, d_out2), lambda i: (i, 0)),
        out_shape=jax.ShapeDtypeStruct((n, d_out2), jnp.float32),
    )(deg, agg, u, w1, b1, w2)


def _tc_out(deg, agg, v, b2, bn):
    n, d = v.shape
    return pl.pallas_call(
        _out_body,
        grid=(n // bn,),
        in_specs=[pl.BlockSpec((NC, bn, 128), lambda i: (0, i, 0)),
                  pl.BlockSpec((NC, bn, d), lambda i: (0, i, 0)),
                  pl.BlockSpec((bn, d), lambda i: (i, 0)),
                  pl.BlockSpec(b2.shape, lambda i: (0, 0))],
        out_specs=pl.BlockSpec((bn, d), lambda i: (i, 0)),
        out_shape=jax.ShapeDtypeStruct((n, d), jnp.float32),
    )(deg, agg, v, b2)


def kernel(x, edge_index, W1, b1, W2, b2):
    n = x.shape[0]
    e = edge_index.shape[1]
    np_ = ((n + 1 + 8 * NS - 1) // (8 * NS)) * (8 * NS)
    grain = NW * CHUNK * GI
    ep = ((e + grain - 1) // grain) * grain
    kch = ep // (NW * CHUNK)

    pad = ep - e
    pad_dst = n + jnp.arange(pad, dtype=jnp.int32) % (np_ - n)
    src = jnp.concatenate(
        [edge_index[0], jnp.zeros((pad,), jnp.int32)]).reshape(NW, kch, CHUNK)
    dst = jnp.concatenate(
        [edge_index[1], pad_dst]).reshape(NW, kch, CHUNK)

    ones_d = jnp.ones((CHUNK, x.shape[1]), jnp.float32)
    zeros_d = jnp.zeros((np_, x.shape[1]), jnp.float32)

    bn = 1000 if n % 1000 == 0 else 8 * (n // 8)

    deg = _sc_degree(dst, ones_d, zeros_d, np_)
    u = _tc_scale(deg, x, bn)
    agg1 = _sc_aggregate(src, dst, u, zeros_d, np_)
    v = _tc_mid(deg, agg1, u, W1, b1.reshape(1, -1), W2, bn)
    agg2 = _sc_aggregate(src, dst, v, zeros_d, np_)
    return _tc_out(deg, agg2, v, b2.reshape(1, -1), bn)


# speedup vs baseline: 2.8536x; 2.8017x over previous
import functools

import jax
import jax.numpy as jnp
from jax import lax
from jax.experimental import pallas as pl
from jax.experimental.pallas import tpu as pltpu
from jax.experimental.pallas import tpu_sc as plsc

NC = 2
NS = 16
NW = NC * NS
CHUNK = 128
NBUF = 4


def _sc_degree(dst3, ones_d, zeros_d, np_):
    kch = dst3.shape[1]
    d = ones_d.shape[1]
    stripe = np_ // NS
    mesh = plsc.VectorSubcoreMesh(core_axis_name="c", subcore_axis_name="s")

    @functools.partial(
        pl.kernel,
        out_type=jax.ShapeDtypeStruct((NC, np_, d), jnp.float32),
        mesh=mesh,
        scratch_types=[
            pltpu.VMEM((kch, CHUNK), jnp.int32),
            pltpu.VMEM((CHUNK, d), jnp.float32),
            pltpu.VMEM_SHARED((np_, d), jnp.float32),
        ],
    )
    def deg_kernel(dst_hbm, ones_hbm, zeros_hbm, out_hbm, dst_v, ones_v, acc):
        c = lax.axis_index("c")
        s = lax.axis_index("s")
        w = c * NS + s
        pltpu.sync_copy(zeros_hbm.at[pl.ds(s * stripe, stripe)],
                        acc.at[pl.ds(s * stripe, stripe)])
        pltpu.sync_copy(dst_hbm.at[w], dst_v)
        pltpu.sync_copy(ones_hbm, ones_v)
        plsc.subcore_barrier()

        def body(k, carry):
            pltpu.sync_copy(ones_v, acc.at[dst_v.at[k]], add=True)
            return carry

        lax.fori_loop(0, kch, body, 0)
        plsc.subcore_barrier()
        pltpu.sync_copy(acc.at[pl.ds(s * stripe, stripe)],
                        out_hbm.at[c, pl.ds(s * stripe, stripe)])

    return deg_kernel(dst3, ones_d, zeros_d)


GI = 16


def _sc_aggregate(src3, dst3, table, zeros_d, np_):
    kch = src3.shape[1]
    d = table.shape[1]
    stripe = np_ // NS
    ng = kch // GI
    mesh = plsc.VectorSubcoreMesh(core_axis_name="c", subcore_axis_name="s")

    @functools.partial(
        pl.kernel,
        out_type=jax.ShapeDtypeStruct((NC, np_, d), jnp.float32),
        mesh=mesh,
        scratch_types=[
            pltpu.VMEM((2, GI, CHUNK), jnp.int32),
            pltpu.VMEM((2, GI, CHUNK), jnp.int32),
            pltpu.VMEM((2, CHUNK, d), jnp.float32),
            pltpu.VMEM_SHARED((np_, d), jnp.float32),
            pltpu.SemaphoreType.DMA((2,)),
            pltpu.SemaphoreType.DMA,
            pltpu.SemaphoreType.DMA,
        ],
    )
    def agg_kernel(src_hbm, dst_hbm, tab_hbm, zeros_hbm, out_hbm,
                   src_v, dst_v, rows_v, acc, gsem, issem, idsem):
        c = lax.axis_index("c")
        s = lax.axis_index("s")
        w = c * NS + s
        pltpu.sync_copy(zeros_hbm.at[pl.ds(s * stripe, stripe)],
                        acc.at[pl.ds(s * stripe, stripe)])
        pltpu.sync_copy(src_hbm.at[w, pl.ds(0, GI)], src_v.at[0])
        pltpu.sync_copy(dst_hbm.at[w, pl.ds(0, GI)], dst_v.at[0])
        plsc.subcore_barrier()

        def group(g, carry):
            gs = lax.rem(g, 2)

            @pl.when(g > 0)
            def _():
                pltpu.make_async_copy(src_hbm.at[w, pl.ds(g * GI, GI)],
                                      src_v.at[gs], issem).wait()
                pltpu.make_async_copy(dst_hbm.at[w, pl.ds(g * GI, GI)],
                                      dst_v.at[gs], idsem).wait()

            @pl.when(g + 1 < ng)
            def _():
                gn = lax.rem(g + 1, 2)
                pltpu.async_copy(src_hbm.at[w, pl.ds((g + 1) * GI, GI)],
                                 src_v.at[gn], issem)
                pltpu.async_copy(dst_hbm.at[w, pl.ds((g + 1) * GI, GI)],
                                 dst_v.at[gn], idsem)

            for b in range(2):
                pltpu.async_copy(tab_hbm.at[src_v.at[gs, b]], rows_v.at[b],
                                 gsem.at[b])

            def pair(j2, carry):
                for b in range(2):
                    j = j2 * 2 + b
                    pltpu.make_async_copy(tab_hbm.at[src_v.at[gs, j]],
                                          rows_v.at[b], gsem.at[b]).wait()
                    pltpu.sync_copy(rows_v.at[b], acc.at[dst_v.at[gs, j]],
                                    add=True)

                    @pl.when(j2 < GI // 2 - 1)
                    def _():
                        pltpu.async_copy(tab_hbm.at[src_v.at[gs, j + 2]],
                                         rows_v.at[b], gsem.at[b])
                return carry

            lax.fori_loop(0, GI // 2, pair, 0)
            return carry

        lax.fori_loop(0, ng, group, 0)
        plsc.subcore_barrier()
        pltpu.sync_copy(acc.at[pl.ds(s * stripe, stripe)],
                        out_hbm.at[c, pl.ds(s * stripe, stripe)])

    return agg_kernel(src3, dst3, table, zeros_d)


def _dis(deg_ref):
    d = deg_ref[0, :, 0:1] + deg_ref[1, :, 0:1] + 1.0
    return lax.rsqrt(d)


def _scale_body(deg_ref, x_ref, u_ref):
    u_ref[...] = x_ref[...] * _dis(deg_ref)


def _mid_body(deg_ref, agg_ref, u_ref, w1_ref, b1_ref, w2_ref, v_ref):
    r = _dis(deg_ref)
    a = (agg_ref[0] + agg_ref[1] + u_ref[...]) * r
    z = lax.dot_general(a, w1_ref[...], (((1,), (0,)), ((), ())),
                        precision=lax.Precision.HIGHEST,
                        preferred_element_type=jnp.float32) + b1_ref[...]
    z = jnp.maximum(z, 0.0)
    h2 = lax.dot_general(z, w2_ref[...], (((1,), (0,)), ((), ())),
                         precision=lax.Precision.HIGHEST,
                         preferred_element_type=jnp.float32)
    v_ref[...] = h2 * r


def _out_body(deg_ref, agg_ref, v_ref, b2_ref, o_ref):
    y = (agg_ref[0] + agg_ref[1] + v_ref[...]) * _dis(deg_ref) + b2_ref[...]
    m = jnp.max(y, axis=1, keepdims=True)
    e = jnp.exp(y - m)
    o_ref[...] = y - m - jnp.log(jnp.sum(e, axis=1, keepdims=True))


def _tc_scale(deg, x, bn):
    n, d = x.shape
    return pl.pallas_call(
        _scale_body,
        grid=(n // bn,),
        in_specs=[pl.BlockSpec((NC, bn, 128), lambda i: (0, i, 0)),
                  pl.BlockSpec((bn, d), lambda i: (i, 0))],
        out_specs=pl.BlockSpec((bn, d), lambda i: (i, 0)),
        out_shape=jax.ShapeDtypeStruct((n, d), jnp.float32),
    )(deg, x)


def _tc_mid(deg, agg, u, w1, b1, w2, bn):
    n, d_in = u.shape
    d_hid, d_out2 = w2.shape
    return pl.pallas_call(
        _mid_body,
        grid=(n // bn,),
        in_specs=[pl.BlockSpec((NC, bn, 128), lambda i: (0, i, 0)),
                  pl.BlockSpec((NC, bn, d_in), lambda i: (0, i, 0)),
                  pl.BlockSpec((bn, d_in), lambda i: (i, 0)),
                  pl.BlockSpec(w1.shape, lambda i: (0, 0)),
                  pl.BlockSpec(b1.shape, lambda i: (0, 0)),
                  pl.BlockSpec(w2.shape, lambda i: (0, 0))],
        out_specs=pl.BlockSpec((bn, d_out2), lambda i: (i, 0)),
        out_shape=jax.ShapeDtypeStruct((n, d_out2), jnp.float32),
    )(deg, agg, u, w1, b1, w2)


def _tc_out(deg, agg, v, b2, bn):
    n, d = v.shape
    return pl.pallas_call(
        _out_body,
        grid=(n // bn,),
        in_specs=[pl.BlockSpec((NC, bn, 128), lambda i: (0, i, 0)),
                  pl.BlockSpec((NC, bn, d), lambda i: (0, i, 0)),
                  pl.BlockSpec((bn, d), lambda i: (i, 0)),
                  pl.BlockSpec(b2.shape, lambda i: (0, 0))],
        out_specs=pl.BlockSpec((bn, d), lambda i: (i, 0)),
        out_shape=jax.ShapeDtypeStruct((n, d), jnp.float32),
    )(deg, agg, v, b2)


def kernel(x, edge_index, W1, b1, W2, b2):
    n = x.shape[0]
    e = edge_index.shape[1]
    np_ = ((n + 1 + 8 * NS - 1) // (8 * NS)) * (8 * NS)
    grain = NW * CHUNK * GI
    ep = ((e + grain - 1) // grain) * grain
    kch = ep // (NW * CHUNK)

    pad = ep - e
    pad_ar = jnp.arange(pad, dtype=jnp.int32)
    src = jnp.concatenate(
        [edge_index[0], pad_ar * 37 % n]).reshape(NW, kch, CHUNK)
    dst = jnp.concatenate(
        [edge_index[1], n + pad_ar % (np_ - n)]).reshape(NW, kch, CHUNK)

    ones_d = jnp.ones((CHUNK, x.shape[1]), jnp.float32)
    zeros_d = jnp.zeros((np_, x.shape[1]), jnp.float32)

    bn = 1000 if n % 1000 == 0 else 8 * (n // 8)

    deg = _sc_degree(dst, ones_d, zeros_d, np_)
    u = _tc_scale(deg, x, bn)
    agg1 = _sc_aggregate(src, dst, u, zeros_d, np_)
    v = _tc_mid(deg, agg1, u, W1, b1.reshape(1, -1), W2, bn)
    agg2 = _sc_aggregate(src, dst, v, zeros_d, np_)
    return _tc_out(deg, agg2, v, b2.reshape(1, -1), bn)


# speedup vs baseline: 3.0095x; 1.0547x over previous
import functools

import jax
import jax.numpy as jnp
from jax import lax
from jax.experimental import pallas as pl
from jax.experimental.pallas import tpu as pltpu
from jax.experimental.pallas import tpu_sc as plsc

NC = 2
NS = 16
NW = NC * NS
CHUNK = 128
NBUF = 4


def _sc_degree(dst3, ones_d, zeros_d, np_):
    kch = dst3.shape[1]
    d = ones_d.shape[1]
    stripe = np_ // NS
    mesh = plsc.VectorSubcoreMesh(core_axis_name="c", subcore_axis_name="s")

    @functools.partial(
        pl.kernel,
        out_type=jax.ShapeDtypeStruct((NC, np_, d), jnp.float32),
        mesh=mesh,
        scratch_types=[
            pltpu.VMEM((kch, CHUNK), jnp.int32),
            pltpu.VMEM((CHUNK, d), jnp.float32),
            pltpu.VMEM_SHARED((np_, d), jnp.float32),
        ],
    )
    def deg_kernel(dst_hbm, ones_hbm, zeros_hbm, out_hbm, dst_v, ones_v, acc):
        c = lax.axis_index("c")
        s = lax.axis_index("s")
        w = c * NS + s
        pltpu.sync_copy(zeros_hbm.at[pl.ds(s * stripe, stripe)],
                        acc.at[pl.ds(s * stripe, stripe)])
        pltpu.sync_copy(dst_hbm.at[w], dst_v)
        pltpu.sync_copy(ones_hbm, ones_v)
        plsc.subcore_barrier()

        def body(k, carry):
            pltpu.sync_copy(ones_v, acc.at[dst_v.at[k]], add=True)
            return carry

        lax.fori_loop(0, kch, body, 0)
        plsc.subcore_barrier()
        pltpu.sync_copy(acc.at[pl.ds(s * stripe, stripe)],
                        out_hbm.at[c, pl.ds(s * stripe, stripe)])

    return deg_kernel(dst3, ones_d, zeros_d)


GI = 16


def _sc_aggregate(src3, dst3, table, zeros_d, np_):
    kch = src3.shape[1]
    d = table.shape[1]
    stripe = np_ // NS
    ng = kch // GI
    mesh = plsc.VectorSubcoreMesh(core_axis_name="c", subcore_axis_name="s")

    @functools.partial(
        pl.kernel,
        out_type=jax.ShapeDtypeStruct((NC, np_, d), jnp.float32),
        mesh=mesh,
        scratch_types=[
            pltpu.VMEM((2, GI, CHUNK), jnp.int32),
            pltpu.VMEM((2, GI, CHUNK), jnp.int32),
            pltpu.VMEM((2, CHUNK, d), jnp.float32),
            pltpu.VMEM_SHARED((np_, d), jnp.float32),
            pltpu.SemaphoreType.DMA((2,)),
            pltpu.SemaphoreType.DMA,
            pltpu.SemaphoreType.DMA,
        ],
    )
    def agg_kernel(src_hbm, dst_hbm, tab_hbm, zeros_hbm, out_hbm,
                   src_v, dst_v, rows_v, acc, gsem, issem, idsem):
        c = lax.axis_index("c")
        s = lax.axis_index("s")
        w = c * NS + s
        pltpu.sync_copy(zeros_hbm.at[pl.ds(s * stripe, stripe)],
                        acc.at[pl.ds(s * stripe, stripe)])
        pltpu.sync_copy(src_hbm.at[w, pl.ds(0, GI)], src_v.at[0])
        pltpu.sync_copy(dst_hbm.at[w, pl.ds(0, GI)], dst_v.at[0])
        plsc.subcore_barrier()

        def group(g, carry):
            gs = lax.rem(g, 2)

            @pl.when(g > 0)
            def _():
                pltpu.make_async_copy(src_hbm.at[w, pl.ds(g * GI, GI)],
                                      src_v.at[gs], issem).wait()
                pltpu.make_async_copy(dst_hbm.at[w, pl.ds(g * GI, GI)],
                                      dst_v.at[gs], idsem).wait()

            @pl.when(g + 1 < ng)
            def _():
                gn = lax.rem(g + 1, 2)
                pltpu.async_copy(src_hbm.at[w, pl.ds((g + 1) * GI, GI)],
                                 src_v.at[gn], issem)
                pltpu.async_copy(dst_hbm.at[w, pl.ds((g + 1) * GI, GI)],
                                 dst_v.at[gn], idsem)

            for b in range(2):
                pltpu.async_copy(tab_hbm.at[src_v.at[gs, b]], rows_v.at[b],
                                 gsem.at[b])

            def pair(j2, carry):
                for b in range(2):
                    j = j2 * 2 + b
                    pltpu.make_async_copy(tab_hbm.at[src_v.at[gs, j]],
                                          rows_v.at[b], gsem.at[b]).wait()
                    pltpu.sync_copy(rows_v.at[b], acc.at[dst_v.at[gs, j]],
                                    add=True)

                    @pl.when(j2 < GI // 2 - 1)
                    def _():
                        pltpu.async_copy(tab_hbm.at[src_v.at[gs, j + 2]],
                                         rows_v.at[b], gsem.at[b])
                return carry

            lax.fori_loop(0, GI // 2, pair, 0)
            return carry

        lax.fori_loop(0, ng, group, 0)
        plsc.subcore_barrier()
        pltpu.sync_copy(acc.at[pl.ds(s * stripe, stripe)],
                        out_hbm.at[c, pl.ds(s * stripe, stripe)])

    return agg_kernel(src3, dst3, table, zeros_d)


def _dis(deg_ref):
    d = deg_ref[0, :, 0:1] + deg_ref[1, :, 0:1] + 1.0
    return lax.rsqrt(d)


def _scale_body(deg_ref, x_ref, u_ref):
    u_ref[...] = x_ref[...] * _dis(deg_ref)


def _mid_body(deg_ref, agg_ref, u_ref, w1_ref, b1_ref, w2_ref, v_ref):
    r = _dis(deg_ref)
    a = (agg_ref[0] + agg_ref[1] + u_ref[...]) * r
    z = lax.dot_general(a, w1_ref[...], (((1,), (0,)), ((), ())),
                        preferred_element_type=jnp.float32) + b1_ref[...]
    z = jnp.maximum(z, 0.0)
    h2 = lax.dot_general(z, w2_ref[...], (((1,), (0,)), ((), ())),
                         preferred_element_type=jnp.float32)
    v_ref[...] = h2 * r


def _out_body(deg_ref, agg_ref, v_ref, b2_ref, o_ref):
    y = (agg_ref[0] + agg_ref[1] + v_ref[...]) * _dis(deg_ref) + b2_ref[...]
    m = jnp.max(y, axis=1, keepdims=True)
    e = jnp.exp(y - m)
    o_ref[...] = y - m - jnp.log(jnp.sum(e, axis=1, keepdims=True))


def _tc_scale(deg, x, bn):
    n, d = x.shape
    return pl.pallas_call(
        _scale_body,
        grid=(n // bn,),
        in_specs=[pl.BlockSpec((NC, bn, 128), lambda i: (0, i, 0)),
                  pl.BlockSpec((bn, d), lambda i: (i, 0))],
        out_specs=pl.BlockSpec((bn, d), lambda i: (i, 0)),
        out_shape=jax.ShapeDtypeStruct((n, d), jnp.float32),
    )(deg, x)


def _tc_mid(deg, agg, u, w1, b1, w2, bn):
    n, d_in = u.shape
    d_hid, d_out2 = w2.shape
    return pl.pallas_call(
        _mid_body,
        grid=(n // bn,),
        in_specs=[pl.BlockSpec((NC, bn, 128), lambda i: (0, i, 0)),
                  pl.BlockSpec((NC, bn, d_in), lambda i: (0, i, 0)),
                  pl.BlockSpec((bn, d_in), lambda i: (i, 0)),
                  pl.BlockSpec(w1.shape, lambda i: (0, 0)),
                  pl.BlockSpec(b1.shape, lambda i: (0, 0)),
                  pl.BlockSpec(w2.shape, lambda i: (0, 0))],
        out_specs=pl.BlockSpec((bn, d_out2), lambda i: (i, 0)),
        out_shape=jax.ShapeDtypeStruct((n, d_out2), jnp.float32),
    )(deg, agg, u, w1, b1, w2)


def _tc_out(deg, agg, v, b2, bn):
    n, d = v.shape
    return pl.pallas_call(
        _out_body,
        grid=(n // bn,),
        in_specs=[pl.BlockSpec((NC, bn, 128), lambda i: (0, i, 0)),
                  pl.BlockSpec((NC, bn, d), lambda i: (0, i, 0)),
                  pl.BlockSpec((bn, d), lambda i: (i, 0)),
                  pl.BlockSpec(b2.shape, lambda i: (0, 0))],
        out_specs=pl.BlockSpec((bn, d), lambda i: (i, 0)),
        out_shape=jax.ShapeDtypeStruct((n, d), jnp.float32),
    )(deg, agg, v, b2)


def kernel(x, edge_index, W1, b1, W2, b2):
    n = x.shape[0]
    e = edge_index.shape[1]
    np_ = ((n + 1 + 8 * NS - 1) // (8 * NS)) * (8 * NS)
    grain = NW * CHUNK * GI
    ep = ((e + grain - 1) // grain) * grain
    kch = ep // (NW * CHUNK)

    pad = ep - e
    pad_ar = jnp.arange(pad, dtype=jnp.int32)
    src = jnp.concatenate(
        [edge_index[0], pad_ar * 37 % n]).reshape(NW, kch, CHUNK)
    dst = jnp.concatenate(
        [edge_index[1], n + pad_ar % (np_ - n)]).reshape(NW, kch, CHUNK)

    ones_d = jnp.ones((CHUNK, x.shape[1]), jnp.float32)
    zeros_d = jnp.zeros((np_, x.shape[1]), jnp.float32)

    bn = 1000 if n % 1000 == 0 else 8 * (n // 8)

    deg = _sc_degree(dst, ones_d, zeros_d, np_)
    u = _tc_scale(deg, x, bn)
    agg1 = _sc_aggregate(src, dst, u, zeros_d, np_)
    v = _tc_mid(deg, agg1, u, W1, b1.reshape(1, -1), W2, bn)
    agg2 = _sc_aggregate(src, dst, v, zeros_d, np_)
    return _tc_out(deg, agg2, v, b2.reshape(1, -1), bn)


# speedup vs baseline: 3.0162x; 1.0022x over previous
import functools

import numpy as np
import jax
import jax.numpy as jnp
from jax import lax
from jax.experimental import pallas as pl
from jax.experimental.pallas import tpu as pltpu
from jax.experimental.pallas import tpu_sc as plsc

NC = 2
NS = 16
NW = NC * NS
CHUNK = 128
NBUF = 4


def _sc_degree(dst3, ones_d, zeros_d, np_):
    kch = dst3.shape[1]
    d = ones_d.shape[1]
    stripe = np_ // NS
    mesh = plsc.VectorSubcoreMesh(core_axis_name="c", subcore_axis_name="s")

    @functools.partial(
        pl.kernel,
        out_type=jax.ShapeDtypeStruct((NC, np_, d), ones_d.dtype),
        mesh=mesh,
        scratch_types=[
            pltpu.VMEM((kch, CHUNK), jnp.int32),
            pltpu.VMEM((CHUNK, d), ones_d.dtype),
            pltpu.VMEM_SHARED((np_, d), ones_d.dtype),
        ],
    )
    def deg_kernel(dst_hbm, ones_hbm, zeros_hbm, out_hbm, dst_v, ones_v, acc):
        c = lax.axis_index("c")
        s = lax.axis_index("s")
        w = c * NS + s
        pltpu.sync_copy(zeros_hbm.at[pl.ds(s * stripe, stripe)],
                        acc.at[pl.ds(s * stripe, stripe)])
        pltpu.sync_copy(dst_hbm.at[w], dst_v)
        pltpu.sync_copy(ones_hbm, ones_v)
        plsc.subcore_barrier()

        def body(k, carry):
            pltpu.sync_copy(ones_v, acc.at[dst_v.at[k]], add=True)
            return carry

        lax.fori_loop(0, kch, body, 0)
        plsc.subcore_barrier()
        pltpu.sync_copy(acc.at[pl.ds(s * stripe, stripe)],
                        out_hbm.at[c, pl.ds(s * stripe, stripe)])

    return deg_kernel(dst3, ones_d, zeros_d)


GI = 16


def _sc_aggregate(src3, dst3, table, zeros_d, np_):
    kch = src3.shape[1]
    d = table.shape[1]
    stripe = np_ // NS
    ng = kch // GI
    mesh = plsc.VectorSubcoreMesh(core_axis_name="c", subcore_axis_name="s")

    @functools.partial(
        pl.kernel,
        out_type=jax.ShapeDtypeStruct((NC, np_, d), jnp.float32),
        mesh=mesh,
        scratch_types=[
            pltpu.VMEM((2, GI, CHUNK), jnp.int32),
            pltpu.VMEM((2, GI, CHUNK), jnp.int32),
            pltpu.VMEM((2, CHUNK, d), jnp.float32),
            pltpu.VMEM_SHARED((np_, d), jnp.float32),
            pltpu.SemaphoreType.DMA((2,)),
            pltpu.SemaphoreType.DMA,
            pltpu.SemaphoreType.DMA,
        ],
    )
    def agg_kernel(src_hbm, dst_hbm, tab_hbm, zeros_hbm, out_hbm,
                   src_v, dst_v, rows_v, acc, gsem, issem, idsem):
        c = lax.axis_index("c")
        s = lax.axis_index("s")
        w = c * NS + s
        pltpu.sync_copy(zeros_hbm.at[pl.ds(s * stripe, stripe)],
                        acc.at[pl.ds(s * stripe, stripe)])
        pltpu.sync_copy(src_hbm.at[w, pl.ds(0, GI)], src_v.at[0])
        pltpu.sync_copy(dst_hbm.at[w, pl.ds(0, GI)], dst_v.at[0])
        plsc.subcore_barrier()

        def group(g, carry):
            gs = lax.rem(g, 2)

            @pl.when(g > 0)
            def _():
                pltpu.make_async_copy(src_hbm.at[w, pl.ds(g * GI, GI)],
                                      src_v.at[gs], issem).wait()
                pltpu.make_async_copy(dst_hbm.at[w, pl.ds(g * GI, GI)],
                                      dst_v.at[gs], idsem).wait()

            @pl.when(g + 1 < ng)
            def _():
                gn = lax.rem(g + 1, 2)
                pltpu.async_copy(src_hbm.at[w, pl.ds((g + 1) * GI, GI)],
                                 src_v.at[gn], issem)
                pltpu.async_copy(dst_hbm.at[w, pl.ds((g + 1) * GI, GI)],
                                 dst_v.at[gn], idsem)

            for b in range(2):
                pltpu.async_copy(tab_hbm.at[src_v.at[gs, b]], rows_v.at[b],
                                 gsem.at[b])

            def pair(j2, carry):
                for b in range(2):
                    j = j2 * 2 + b
                    pltpu.make_async_copy(tab_hbm.at[src_v.at[gs, j]],
                                          rows_v.at[b], gsem.at[b]).wait()
                    pltpu.sync_copy(rows_v.at[b], acc.at[dst_v.at[gs, j]],
                                    add=True)

                    @pl.when(j2 < GI // 2 - 1)
                    def _():
                        pltpu.async_copy(tab_hbm.at[src_v.at[gs, j + 2]],
                                         rows_v.at[b], gsem.at[b])
                return carry

            lax.fori_loop(0, GI // 2, pair, 0)
            return carry

        lax.fori_loop(0, ng, group, 0)
        plsc.subcore_barrier()
        pltpu.sync_copy(acc.at[pl.ds(s * stripe, stripe)],
                        out_hbm.at[c, pl.ds(s * stripe, stripe)])

    return agg_kernel(src3, dst3, table, zeros_d)


def _dis(deg_ref):
    d0 = deg_ref[0, :, 0:1].astype(jnp.float32)
    d1 = deg_ref[1, :, 0:1].astype(jnp.float32)
    return lax.rsqrt(d0 + d1 + 1.0)


def _scale_body(deg_ref, x_ref, u_ref):
    u_ref[...] = x_ref[...] * _dis(deg_ref)


def _mid_body(deg_ref, agg_ref, u_ref, w1_ref, b1_ref, w2_ref, v_ref):
    r = _dis(deg_ref)
    a = (agg_ref[0] + agg_ref[1] + u_ref[...]) * r
    z = lax.dot_general(a, w1_ref[...], (((1,), (0,)), ((), ())),
                        preferred_element_type=jnp.float32) + b1_ref[...]
    z = jnp.maximum(z, 0.0)
    h2 = lax.dot_general(z, w2_ref[...], (((1,), (0,)), ((), ())),
                         preferred_element_type=jnp.float32)
    v_ref[...] = h2 * r


def _out_body(deg_ref, agg_ref, v_ref, b2_ref, o_ref):
    y = (agg_ref[0] + agg_ref[1] + v_ref[...]) * _dis(deg_ref) + b2_ref[...]
    m = jnp.max(y, axis=1, keepdims=True)
    e = jnp.exp(y - m)
    o_ref[...] = y - m - jnp.log(jnp.sum(e, axis=1, keepdims=True))


def _tc_scale(deg, x, bn):
    n, d = x.shape
    return pl.pallas_call(
        _scale_body,
        grid=(n // bn,),
        in_specs=[pl.BlockSpec((NC, bn, 128), lambda i: (0, i, 0)),
                  pl.BlockSpec((bn, d), lambda i: (i, 0))],
        out_specs=pl.BlockSpec((bn, d), lambda i: (i, 0)),
        out_shape=jax.ShapeDtypeStruct((n, d), jnp.float32),
    )(deg, x)


def _tc_mid(deg, agg, u, w1, b1, w2, bn):
    n, d_in = u.shape
    d_hid, d_out2 = w2.shape
    return pl.pallas_call(
        _mid_body,
        grid=(n // bn,),
        in_specs=[pl.BlockSpec((NC, bn, 128), lambda i: (0, i, 0)),
                  pl.BlockSpec((NC, bn, d_in), lambda i: (0, i, 0)),
                  pl.BlockSpec((bn, d_in), lambda i: (i, 0)),
                  pl.BlockSpec(w1.shape, lambda i: (0, 0)),
                  pl.BlockSpec(b1.shape, lambda i: (0, 0)),
                  pl.BlockSpec(w2.shape, lambda i: (0, 0))],
        out_specs=pl.BlockSpec((bn, d_out2), lambda i: (i, 0)),
        out_shape=jax.ShapeDtypeStruct((n, d_out2), jnp.float32),
    )(deg, agg, u, w1, b1, w2)


def _tc_out(deg, agg, v, b2, bn):
    n, d = v.shape
    return pl.pallas_call(
        _out_body,
        grid=(n // bn,),
        in_specs=[pl.BlockSpec((NC, bn, 128), lambda i: (0, i, 0)),
                  pl.BlockSpec((NC, bn, d), lambda i: (0, i, 0)),
                  pl.BlockSpec((bn, d), lambda i: (i, 0)),
                  pl.BlockSpec(b2.shape, lambda i: (0, 0))],
        out_specs=pl.BlockSpec((bn, d), lambda i: (i, 0)),
        out_shape=jax.ShapeDtypeStruct((n, d), jnp.float32),
    )(deg, agg, v, b2)


def kernel(x, edge_index, W1, b1, W2, b2):
    n = x.shape[0]
    e = edge_index.shape[1]
    np_ = ((n + 1 + 8 * NS - 1) // (8 * NS)) * (8 * NS)
    grain = NW * CHUNK * GI
    ep = ((e + grain - 1) // grain) * grain
    kch = ep // (NW * CHUNK)

    pad = ep - e
    pad_ar = np.arange(pad, dtype=np.int32)
    pad_src = jnp.asarray(pad_ar * 37 % n)
    pad_dst = jnp.asarray(n + pad_ar % (np_ - n))
    src = jnp.concatenate([edge_index[0], pad_src]).reshape(NW, kch, CHUNK)
    dst = jnp.concatenate([edge_index[1], pad_dst]).reshape(NW, kch, CHUNK)

    ones_d = jnp.ones((CHUNK, x.shape[1]), jnp.float32)
    zeros_d = jnp.zeros((np_, x.shape[1]), jnp.float32)

    bn = 1000 if n % 1000 == 0 else 8 * (n // 8)

    deg = _sc_degree(dst, ones_d, zeros_d, np_)
    u = _tc_scale(deg, x, bn)
    agg1 = _sc_aggregate(src, dst, u, zeros_d, np_)
    v = _tc_mid(deg, agg1, u, W1, b1.reshape(1, -1), W2, bn)
    agg2 = _sc_aggregate(src, dst, v, zeros_d, np_)
    return _tc_out(deg, agg2, v, b2.reshape(1, -1), bn)
